# Initial kernel scaffold; baseline (speedup 1.0000x reference)
import jax
import jax.numpy as jnp
from jax.experimental import pallas as pl
from jax.experimental.pallas import tpu as pltpu

P_DEG = 2
DIM = 3


def _build_a_body(nu_ref, idx_ref, a_ref):
    m = jax.lax.broadcasted_iota(jnp.int32, a_ref.shape, 1)
    i0 = idx_ref[...] - P_DEG
    acc = jnp.zeros(a_ref.shape, jnp.float32)
    for j in range(P_DEG + 1):
        acc = acc + jnp.where(m == i0 + j, nu_ref[:, j:j + 1], 0.0)
    a_ref[...] = acc


def _blend_body(a_ref, xtt_ref, out_ref):
    a = a_ref[...]
    c = [jnp.dot(a, xtt_ref[d], preferred_element_type=jnp.float32)
         for d in range(DIM + 1)]
    inv = 1.0 / c[DIM]
    out_ref[...] = jnp.stack([c[d] * inv for d in range(DIM)], axis=0)


def kernel(input, Nu, uspan):
    B, M, D1 = input.shape
    O = Nu.shape[0]
    idx2d = uspan.astype(jnp.int32).reshape(O, 1)
    xtt = jnp.transpose(input, (2, 1, 0))

    a = pl.pallas_call(
        _build_a_body,
        out_shape=jax.ShapeDtypeStruct((O, M), jnp.float32),
    )(Nu, idx2d)

    BT = 128
    out3 = pl.pallas_call(
        _blend_body,
        grid=(B // BT,),
        in_specs=[
            pl.BlockSpec((O, M), lambda i: (0, 0)),
            pl.BlockSpec((D1, M, BT), lambda i: (0, 0, i)),
        ],
        out_specs=pl.BlockSpec((DIM, O, BT), lambda i: (0, 0, i)),
        out_shape=jax.ShapeDtypeStruct((DIM, O, B), jnp.float32),
    )(a, xtt)

    return jnp.transpose(out3, (2, 1, 0))


# speedup vs baseline: 14.3525x; 14.3525x over previous
import jax
import jax.numpy as jnp
from jax.experimental import pallas as pl
from jax.experimental.pallas import tpu as pltpu

P_DEG = 2
DIM = 3


def _build_at_body(nut_ref, idx_ref, at_ref):
    m = jax.lax.broadcasted_iota(jnp.int32, at_ref.shape, 0)
    i0 = idx_ref[...] - P_DEG
    acc = jnp.zeros(at_ref.shape, jnp.float32)
    for j in range(P_DEG + 1):
        acc = acc + jnp.where(m == i0 + j, nut_ref[j:j + 1, :], 0.0)
    at_ref[...] = acc


def _blend_body(at_ref, xtt_ref, out_ref):
    at = at_ref[...]
    c = [jnp.dot(xtt_ref[d], at, preferred_element_type=jnp.float32)
         for d in range(DIM + 1)]
    inv = 1.0 / c[DIM]
    out_ref[...] = jnp.stack([c[d] * inv for d in range(DIM)], axis=0)


def kernel(input, Nu, uspan):
    B, M, D1 = input.shape
    O = Nu.shape[0]
    idx_row = uspan.astype(jnp.int32).reshape(1, O)
    nut = jnp.transpose(Nu, (1, 0))
    xtt = jnp.transpose(input, (2, 0, 1))

    at = pl.pallas_call(
        _build_at_body,
        out_shape=jax.ShapeDtypeStruct((M, O), jnp.float32),
    )(nut, idx_row)

    BT = 128
    out3 = pl.pallas_call(
        _blend_body,
        grid=(B // BT,),
        in_specs=[
            pl.BlockSpec((M, O), lambda i: (0, 0)),
            pl.BlockSpec((D1, BT, M), lambda i: (0, i, 0)),
        ],
        out_specs=pl.BlockSpec((DIM, BT, O), lambda i: (0, i, 0)),
        out_shape=jax.ShapeDtypeStruct((DIM, B, O), jnp.float32),
    )(at, xtt)

    return jnp.transpose(out3, (1, 2, 0))
